# Initial kernel scaffold; baseline (speedup 1.0000x reference)
import jax
import jax.numpy as jnp
from jax.experimental import pallas as pl


def kernel(x, W1, W2, W3, Wr):
    raise NotImplementedError("write your pallas kernel here")


# speedup vs baseline: 2.4991x; 2.4991x over previous
import functools

import jax
import jax.numpy as jnp
from jax.experimental import pallas as pl
from jax.experimental.pallas import tpu as pltpu

N_EMBD = 256
N_EXPERTS = 8
HIDDEN = 682
HID_PAD = 768


def _moe_block_kernel(x_ref, wr_ref, w1_ref, w3_ref, w2_ref, out_ref):
    x = x_ref[...]
    logits = jax.lax.dot_general(
        x, wr_ref[...], (((1,), (1,)), ((), ())),
        preferred_element_type=jnp.float32)
    m = jnp.max(logits, axis=-1, keepdims=True)
    unnorm = jnp.exp(logits - m)
    probs = unnorm / jnp.sum(unnorm, axis=-1, keepdims=True)
    eidx = jax.lax.broadcasted_iota(jnp.int32, probs.shape, 1)
    p1 = jnp.max(probs, axis=-1, keepdims=True)
    i1 = jnp.min(jnp.where(probs == p1, eidx, N_EXPERTS), axis=-1,
                 keepdims=True)
    probs_m = jnp.where(eidx == i1, -1.0, probs)
    p2 = jnp.max(probs_m, axis=-1, keepdims=True)
    i2 = jnp.min(jnp.where(probs_m == p2, eidx, N_EXPERTS), axis=-1,
                 keepdims=True)
    denom = p1 + p2 + 1e-9
    w = (jnp.where(eidx == i1, p1, 0.0)
         + jnp.where(eidx == i2, p2, 0.0)) / denom

    acc = jnp.zeros_like(x)
    for e in range(N_EXPERTS):
        g = jax.lax.dot_general(
            x, w1_ref[e], (((1,), (1,)), ((), ())),
            preferred_element_type=jnp.float32)
        u = jax.lax.dot_general(
            x, w3_ref[e], (((1,), (1,)), ((), ())),
            preferred_element_type=jnp.float32)
        h = (g * jax.nn.sigmoid(g)) * u
        ex = jax.lax.dot_general(
            h, w2_ref[e], (((1,), (1,)), ((), ())),
            preferred_element_type=jnp.float32)
        acc = acc + w[:, e:e + 1] * ex
    out_ref[...] = acc


def kernel(x, W1, W2, W3, Wr):
    B, T, C = x.shape
    flat = x.reshape(-1, C)
    n_tok = flat.shape[0]
    pad = HID_PAD - HIDDEN
    W1p = jnp.pad(W1, ((0, 0), (0, pad), (0, 0)))
    W3p = jnp.pad(W3, ((0, 0), (0, pad), (0, 0)))
    W2p = jnp.pad(W2, ((0, 0), (0, 0), (0, pad)))

    TB = 512
    grid = (n_tok // TB,)
    out = pl.pallas_call(
        _moe_block_kernel,
        grid=grid,
        in_specs=[
            pl.BlockSpec((TB, C), lambda i: (i, 0)),
            pl.BlockSpec((N_EXPERTS, C), lambda i: (0, 0)),
            pl.BlockSpec((N_EXPERTS, HID_PAD, C), lambda i: (0, 0, 0)),
            pl.BlockSpec((N_EXPERTS, HID_PAD, C), lambda i: (0, 0, 0)),
            pl.BlockSpec((N_EXPERTS, C, HID_PAD), lambda i: (0, 0, 0)),
        ],
        out_specs=pl.BlockSpec((TB, C), lambda i: (i, 0)),
        out_shape=jax.ShapeDtypeStruct((n_tok, C), x.dtype),
        compiler_params=pltpu.CompilerParams(
            dimension_semantics=("arbitrary",),
        ),
    )(flat, Wr, W1p, W3p, W2p)
    return out.reshape(B, T, C)


# speedup vs baseline: 2.7378x; 1.0955x over previous
import functools

import jax
import jax.numpy as jnp
from jax.experimental import pallas as pl
from jax.experimental.pallas import tpu as pltpu

N_EMBD = 256
N_EXPERTS = 8
HIDDEN = 682
HID_PAD = 768


def _moe_block_kernel(x_ref, wr_ref, w1_ref, w3_ref, w2_ref, out_ref):
    x = x_ref[...]
    logits = jax.lax.dot_general(
        x, wr_ref[...], (((1,), (1,)), ((), ())),
        preferred_element_type=jnp.float32)
    m = jnp.max(logits, axis=-1, keepdims=True)
    unnorm = jnp.exp(logits - m)
    probs = unnorm / jnp.sum(unnorm, axis=-1, keepdims=True)
    eidx = jax.lax.broadcasted_iota(jnp.int32, probs.shape, 1)
    p1 = jnp.max(probs, axis=-1, keepdims=True)
    i1 = jnp.min(jnp.where(probs == p1, eidx, N_EXPERTS), axis=-1,
                 keepdims=True)
    probs_m = jnp.where(eidx == i1, -1.0, probs)
    p2 = jnp.max(probs_m, axis=-1, keepdims=True)
    i2 = jnp.min(jnp.where(probs_m == p2, eidx, N_EXPERTS), axis=-1,
                 keepdims=True)
    denom = p1 + p2 + 1e-9
    w = (jnp.where(eidx == i1, p1, 0.0)
         + jnp.where(eidx == i2, p2, 0.0)) / denom

    xb = x.astype(jnp.bfloat16)
    acc = jnp.zeros_like(x)
    for e in range(N_EXPERTS):
        g = jax.lax.dot_general(
            xb, w1_ref[e], (((1,), (1,)), ((), ())),
            preferred_element_type=jnp.float32)
        u = jax.lax.dot_general(
            xb, w3_ref[e], (((1,), (1,)), ((), ())),
            preferred_element_type=jnp.float32)
        h = (g * jax.nn.sigmoid(g)) * u
        ex = jax.lax.dot_general(
            h.astype(jnp.bfloat16), w2_ref[e], (((1,), (1,)), ((), ())),
            preferred_element_type=jnp.float32)
        acc = acc + w[:, e:e + 1] * ex
    out_ref[...] = acc


def kernel(x, W1, W2, W3, Wr):
    B, T, C = x.shape
    flat = x.reshape(-1, C)
    n_tok = flat.shape[0]
    pad = HID_PAD - HIDDEN
    W1p = jnp.pad(W1, ((0, 0), (0, pad), (0, 0))).astype(jnp.bfloat16)
    W3p = jnp.pad(W3, ((0, 0), (0, pad), (0, 0))).astype(jnp.bfloat16)
    W2p = jnp.pad(W2, ((0, 0), (0, 0), (0, pad))).astype(jnp.bfloat16)

    TB = 512
    grid = (n_tok // TB,)
    out = pl.pallas_call(
        _moe_block_kernel,
        grid=grid,
        in_specs=[
            pl.BlockSpec((TB, C), lambda i: (i, 0)),
            pl.BlockSpec((N_EXPERTS, C), lambda i: (0, 0)),
            pl.BlockSpec((N_EXPERTS, HID_PAD, C), lambda i: (0, 0, 0)),
            pl.BlockSpec((N_EXPERTS, HID_PAD, C), lambda i: (0, 0, 0)),
            pl.BlockSpec((N_EXPERTS, C, HID_PAD), lambda i: (0, 0, 0)),
        ],
        out_specs=pl.BlockSpec((TB, C), lambda i: (i, 0)),
        out_shape=jax.ShapeDtypeStruct((n_tok, C), x.dtype),
        compiler_params=pltpu.CompilerParams(
            dimension_semantics=("arbitrary",),
        ),
    )(flat, Wr, W1p, W3p, W2p)
    return out.reshape(B, T, C)
